# baseline (speedup 1.0000x reference)
import functools

import jax
import jax.numpy as jnp
from jax import lax
from jax.experimental import pallas as pl
from jax.experimental.pallas import tpu as pltpu
from jax.experimental.pallas import tpu_sc as plsc

_NUM_INST = 1000000
_CHANNELS = 16
_BATCH = 16384


def _make_gather(num_rows, channels, batch):
    info = plsc.get_sparse_core_info()
    nc, ns = info.num_cores, info.num_subcores
    nw = nc * ns
    assert batch % (8 * nw) == 0
    b_per_w = batch // nw
    mesh = plsc.VectorSubcoreMesh(core_axis_name="c", subcore_axis_name="s")

    @functools.partial(
        pl.kernel,
        mesh=mesh,
        out_type=jax.ShapeDtypeStruct((batch, channels), jnp.float32),
        scratch_types=[
            pltpu.VMEM((b_per_w,), jnp.int32),
            pltpu.VMEM((b_per_w, channels), jnp.float32),
            pltpu.SemaphoreType.DMA,
        ],
        compiler_params=pltpu.CompilerParams(use_tc_tiling_on_sc=False),
    )
    def gather_kernel(table_hbm, idx_hbm, out_hbm, idx_v, rows_v, sem):
        wid = lax.axis_index("s") * nc + lax.axis_index("c")
        base = wid * b_per_w
        pltpu.sync_copy(idx_hbm.at[pl.ds(base, b_per_w)], idx_v)
        pltpu.async_copy(table_hbm.at[idx_v], rows_v, sem).wait()
        pltpu.sync_copy(rows_v, out_hbm.at[pl.ds(base, b_per_w)])

    return gather_kernel


def kernel(inst_id, mapping_weight):
    gather = _make_gather(_NUM_INST, _CHANNELS, _BATCH)
    return gather(mapping_weight, inst_id.astype(jnp.int32))


# speedup vs baseline: 6.1119x; 6.1119x over previous
import functools

import jax
import jax.numpy as jnp
from jax import lax
from jax.experimental import pallas as pl
from jax.experimental.pallas import tpu as pltpu
from jax.experimental.pallas import tpu_sc as plsc

_NUM_INST = 1000000
_CHANNELS = 16
_BATCH = 16384
_G = 16
_NBUF = 2


def _make_gather(channels, batch):
    info = plsc.get_sparse_core_info()
    nc, ns = info.num_cores, info.num_subcores
    nw = nc * ns
    assert batch % (8 * nw) == 0
    b_per_w = batch // nw
    n_groups = b_per_w // _G
    mesh = plsc.VectorSubcoreMesh(core_axis_name="c", subcore_axis_name="s")

    @functools.partial(
        pl.kernel,
        mesh=mesh,
        out_type=jax.ShapeDtypeStruct((channels, batch), jnp.float32),
        scratch_types=[
            pltpu.VMEM((b_per_w,), jnp.int32),
            pltpu.VMEM((_NBUF, _G, 2, 8, 128), jnp.float32),
            pltpu.VMEM((channels, b_per_w), jnp.float32),
            [pltpu.SemaphoreType.DMA] * _NBUF,
        ],
        compiler_params=pltpu.CompilerParams(needs_layout_passes=False),
    )
    def gather_kernel(tab_hbm, idx_hbm, out_hbm, idx_v, blk_v, out_v, sems):
        wid = lax.axis_index("s") * nc + lax.axis_index("c")
        base = wid * b_per_w
        pltpu.sync_copy(idx_hbm.at[pl.ds(base, b_per_w)], idx_v)
        iota = lax.iota(jnp.int32, 16)
        ct = iota >> 3
        cs = iota & 7

        def load_group(g):
            return idx_v[pl.ds(pl.multiple_of(g * _G, _G), _G)]

        def issue(g, slot):
            cols = (load_group(g) >> 7) * 128
            for s in range(_G):
                col = pl.multiple_of(cols[s], 128)
                pltpu.async_copy(
                    tab_hbm.at[:, :, pl.ds(col, 128)],
                    blk_v.at[slot, s],
                    sems[slot],
                )

        def consume(g, slot):
            lanes = load_group(g) & 127
            for s in range(_G):
                pltpu.make_async_copy(
                    tab_hbm.at[:, :, pl.ds(0, 128)],
                    blk_v.at[slot, s],
                    sems[slot],
                ).wait()
                lane = jnp.full((16,), lanes[s], jnp.int32)
                vals = plsc.load_gather(blk_v.at[slot, s], [ct, cs, lane])
                plsc.store_scatter(
                    out_v, [iota, jnp.full((16,), g * _G + s, jnp.int32)], vals
                )

        for b in range(_NBUF):
            issue(jnp.int32(b), b)

        def group(g2, _):
            for b in range(_NBUF):
                g = g2 * _NBUF + b
                consume(g, b)
                issue(g + _NBUF, b)
            return _

        lax.fori_loop(0, n_groups // _NBUF - 1, group, None)
        for b in range(_NBUF):
            consume(jnp.int32(n_groups - _NBUF + b), b)
        pltpu.sync_copy(out_v, out_hbm.at[:, pl.ds(base, b_per_w)])

    return gather_kernel


def kernel(inst_id, mapping_weight):
    gather = _make_gather(_CHANNELS, _BATCH)
    table = mapping_weight.T.reshape(2, 8, _NUM_INST)
    out_t = gather(table, inst_id.astype(jnp.int32))
    return out_t.T
